# baseline (speedup 1.0000x reference)
import jax
import jax.numpy as jnp
from jax.experimental import pallas as pl

N = 10000
E = 320000
D = 128
HID = 32
H1 = 2
G = 64


def _seg_softmax(logits, seg, num_segments):
    m = jax.ops.segment_max(logits, seg, num_segments=num_segments)
    m = jnp.where(jnp.isfinite(m), m, 0.0)
    l = logits - jax.lax.stop_gradient(m[seg])
    e = jnp.exp(l)
    s = jax.ops.segment_sum(e, seg, num_segments=num_segments)
    return e / (s[seg] + 1e-16)


def _gat(x, src, dst, Wl, bl, Wr, br, att, bias, heads, out_ch, n_nodes):
    xl = (x @ Wl + bl).reshape(n_nodes, heads, out_ch)
    xr = (x @ Wr + br).reshape(n_nodes, heads, out_ch)
    xj = xl[src]
    xi = xr[dst]
    e = jax.nn.leaky_relu(xi + xj, negative_slope=0.2)
    logits = jnp.sum(e * att[None, :, :], axis=-1)
    alpha = _seg_softmax(logits, dst, n_nodes)
    out = jax.ops.segment_sum(xj * alpha[:, :, None], dst, num_segments=n_nodes)
    out = out.reshape(n_nodes, heads * out_ch)
    return out + bias


def kernel(x, edge_index, batch, Wl1, bl1, Wr1, br1, att1, bias1,
           Wl2, bl2, Wr2, br2, att2, bias2,
           Wg1, bg1, Wg2, bg2, W1, b1, W2, b2):
    loop = jnp.arange(N, dtype=edge_index.dtype)
    src = jnp.concatenate([edge_index[0], loop])
    dst = jnp.concatenate([edge_index[1], loop])
    h = jax.nn.relu(_gat(x, src, dst, Wl1, bl1, Wr1, br1, att1, bias1, H1, HID, N))
    h = jax.nn.relu(_gat(h, src, dst, Wl2, bl2, Wr2, br2, att2, bias2, 1, HID, N))
    gate = jax.nn.relu(h @ Wg1 + bg1) @ Wg2 + bg2
    gate = _seg_softmax(gate[:, 0], batch, G)
    pooled = jax.ops.segment_sum(gate[:, None] * h, batch, num_segments=G)
    out = jax.nn.relu(pooled @ W1 + b1)
    out = (out @ W2 + b2).squeeze(-1)
    return out


# speedup vs baseline: 40.2575x; 40.2575x over previous
import functools

import jax
import jax.numpy as jnp
from jax import lax
from jax.experimental import pallas as pl
from jax.experimental.pallas import tpu as pltpu
from jax.experimental.pallas import tpu_sc as plsc

N = 10000
E = 320000
D = 128
HID = 32
H1 = 2
G = 64

NP = 10240
EV = E + N
EP = 331776
NTILES = 32
EPT = EP // NTILES
K = 128
NCHUNKS = EPT // K
ROWS_PT = NP // 16

_f32 = jnp.float32
_i32 = jnp.int32


def _lane_take(v, idx):
    return lax.gather(
        v, idx[:, None],
        lax.GatherDimensionNumbers(
            offset_dims=(), collapsed_slice_dims=(0,), start_index_map=(0,)),
        slice_sizes=(1,),
        mode=lax.GatherScatterMode.PROMISE_IN_BOUNDS)


def _make_sc_edge(heads, outw):
    F = heads * HID

    mesh = plsc.VectorSubcoreMesh(
        core_axis_name="c", subcore_axis_name="s", num_cores=2,
        num_subcores=16)

    def body(src_hbm, dst_hbm, xl_hbm, xr_hbm, att_hbm, out_hbm,
             srcv, dstv, xj, xi, outb, attv, acc, sem1, sem2):
        c = lax.axis_index("c")
        s = lax.axis_index("s")
        lane = lax.iota(_i32, 16)

        pltpu.sync_copy(att_hbm, attv)

        def zrow(r, _):
            for q in range(outw // 16):
                outb[r, pl.ds(q * 16, 16)] = jnp.zeros((16,), _f32)
            return 0
        lax.fori_loop(0, K, zrow, 0)

        def zacc(j, _):
            pltpu.sync_copy(outb, acc.at[pl.ds(s * ROWS_PT + j * K, K)])
            return 0
        lax.fori_loop(0, ROWS_PT // K, zacc, 0)
        plsc.subcore_barrier()

        wbase = (c * 16 + s) * EPT

        def chunk(k, _):
            base = wbase + k * K
            pltpu.sync_copy(src_hbm.at[pl.ds(base, K)], srcv)
            pltpu.sync_copy(dst_hbm.at[pl.ds(base, K)], dstv)
            cp1 = pltpu.async_copy(xl_hbm.at[srcv], xj, sem1)
            cp2 = pltpu.async_copy(xr_hbm.at[dstv], xi, sem2)
            cp1.wait()
            cp2.wait()

            av = [attv[pl.ds(q * 16, 16)] for q in range(F // 16)]
            qph = HID // 16
            lanef = lane.astype(_f32)

            def edge(i, _):
                for u in range(4):
                    e = i * 4 + u
                    vjs = [xj[e, pl.ds(q * 16, 16)] for q in range(F // 16)]
                    hsum = [jnp.zeros((16,), _f32) for _ in range(heads)]
                    for q in range(F // 16):
                        t = vjs[q] + xi[e, pl.ds(q * 16, 16)]
                        lr = jnp.maximum(t, 0.2 * t)
                        hsum[q // qph] = hsum[q // qph] + lr * av[q]
                    vf = jnp.clip(
                        jnp.full((16,), EV - (base + e), _i32).astype(_f32),
                        0.0, 1.0)
                    tail = jnp.zeros((16,), _f32)
                    for h in range(heads):
                        sv = hsum[h]
                        for kk in (8, 4, 2, 1):
                            sv = sv + _lane_take(sv, lane ^ kk)
                        pv = jnp.exp(sv) * vf
                        for q in range(h * qph, (h + 1) * qph):
                            outb[e, pl.ds(q * 16, 16)] = vjs[q] * pv
                        ohv = jnp.maximum(
                            1.0 - jnp.abs(lanef - float(h)), 0.0)
                        tail = tail + pv * ohv
                    outb[e, pl.ds(F, 16)] = tail
                return 0
            lax.fori_loop(0, K // 4, edge, 0)

            pltpu.sync_copy(outb, acc.at[dstv], add=True)
            return 0
        lax.fori_loop(0, NCHUNKS, chunk, 0)

        plsc.subcore_barrier()

        def wout(j, _):
            r = s * ROWS_PT + j * K
            pltpu.sync_copy(acc.at[pl.ds(r, K)], out_hbm.at[c, pl.ds(r, K)])
            return 0
        lax.fori_loop(0, ROWS_PT // K, wout, 0)

    return pl.kernel(
        body,
        out_type=jax.ShapeDtypeStruct((2, NP, outw), _f32),
        mesh=mesh,
        compiler_params=pltpu.CompilerParams(use_tc_tiling_on_sc=False),
        scratch_types=[
            pltpu.VMEM((K,), _i32),
            pltpu.VMEM((K,), _i32),
            pltpu.VMEM((K, F), _f32),
            pltpu.VMEM((K, F), _f32),
            pltpu.VMEM((K, outw), _f32),
            pltpu.VMEM((F,), _f32),
            pltpu.VMEM_SHARED((NP, outw), _f32),
            pltpu.SemaphoreType.DMA,
            pltpu.SemaphoreType.DMA,
        ],
    )


_sc_edge_l1 = _make_sc_edge(H1, 80)
_sc_edge_l2 = _make_sc_edge(1, 48)

_BLK = 512
_NBLK = NP // _BLK


def _mm1_body(x_ref, wl_ref, bl_ref, wr_ref, br_ref, xl_ref, xr_ref):
    xb = x_ref[...]
    xl_ref[...] = jnp.dot(xb, wl_ref[...],
                          preferred_element_type=_f32) + bl_ref[...]
    xr_ref[...] = jnp.dot(xb, wr_ref[...],
                          preferred_element_type=_f32) + br_ref[...]


def _mm1(xp, Wl1, bl1, Wr1, br1):
    return pl.pallas_call(
        _mm1_body,
        grid=(_NBLK,),
        in_specs=[
            pl.BlockSpec((_BLK, D), lambda i: (i, 0)),
            pl.BlockSpec((D, H1 * HID), lambda i: (0, 0)),
            pl.BlockSpec((1, H1 * HID), lambda i: (0, 0)),
            pl.BlockSpec((D, H1 * HID), lambda i: (0, 0)),
            pl.BlockSpec((1, H1 * HID), lambda i: (0, 0)),
        ],
        out_specs=[
            pl.BlockSpec((_BLK, H1 * HID), lambda i: (i, 0)),
            pl.BlockSpec((_BLK, H1 * HID), lambda i: (i, 0)),
        ],
        out_shape=[
            jax.ShapeDtypeStruct((NP, H1 * HID), _f32),
            jax.ShapeDtypeStruct((NP, H1 * HID), _f32),
        ],
    )(xp, Wl1, bl1, Wr1, br1)


def _fin1_body(acc_ref, b1_ref, wl_ref, bl_ref, wr_ref, br_ref,
               xl_ref, xr_ref):
    a = acc_ref[0] + acc_ref[1]
    s0 = a[:, 64:65] + 1e-16
    s1 = a[:, 65:66] + 1e-16
    h = jnp.concatenate([a[:, 0:32] / s0, a[:, 32:64] / s1], axis=1)
    h = jax.nn.relu(h + b1_ref[...])
    xl_ref[...] = jnp.dot(h, wl_ref[...],
                          preferred_element_type=_f32) + bl_ref[...]
    xr_ref[...] = jnp.dot(h, wr_ref[...],
                          preferred_element_type=_f32) + br_ref[...]


def _fin1(acc, bias1, Wl2, bl2, Wr2, br2):
    return pl.pallas_call(
        _fin1_body,
        grid=(_NBLK,),
        in_specs=[
            pl.BlockSpec((2, _BLK, 80), lambda i: (0, i, 0)),
            pl.BlockSpec((1, H1 * HID), lambda i: (0, 0)),
            pl.BlockSpec((H1 * HID, HID), lambda i: (0, 0)),
            pl.BlockSpec((1, HID), lambda i: (0, 0)),
            pl.BlockSpec((H1 * HID, HID), lambda i: (0, 0)),
            pl.BlockSpec((1, HID), lambda i: (0, 0)),
        ],
        out_specs=[
            pl.BlockSpec((_BLK, HID), lambda i: (i, 0)),
            pl.BlockSpec((_BLK, HID), lambda i: (i, 0)),
        ],
        out_shape=[
            jax.ShapeDtypeStruct((NP, HID), _f32),
            jax.ShapeDtypeStruct((NP, HID), _f32),
        ],
    )(acc, bias1, Wl2, bl2, Wr2, br2)


def _fin2_body(acc_ref, batch_ref, b2_ref, wg1_ref, bg1_ref, wg2_ref,
               bg2_ref, w1_ref, b1_ref, w2_ref, b2f_ref, out_ref,
               pooled_ref):
    i = pl.program_id(0)

    @pl.when(i == 0)
    def _():
        pooled_ref[...] = jnp.zeros((G, 48), _f32)

    a = acc_ref[0] + acc_ref[1]
    s = a[:, 32:33] + 1e-16
    h2 = jax.nn.relu(a[:, 0:32] / s + b2_ref[...])
    gate = jnp.dot(jax.nn.relu(jnp.dot(h2, wg1_ref[...],
                                       preferred_element_type=_f32)
                               + bg1_ref[...]),
                   wg2_ref[...], preferred_element_type=_f32) + bg2_ref[...]
    w = jnp.exp(gate[:, 0])
    bidx = batch_ref[0, 0, :].astype(_f32)
    gi = lax.broadcasted_iota(_i32, (G, _BLK), 0).astype(_f32)
    oh = jnp.maximum(1.0 - jnp.abs(gi - bidx[None, :]), 0.0)
    ohw = oh * w[None, :]
    feat = jnp.concatenate(
        [h2, jnp.ones((_BLK, 1), _f32), jnp.zeros((_BLK, 15), _f32)], axis=1)
    pooled_ref[...] += jnp.dot(ohw, feat, preferred_element_type=_f32)

    @pl.when(i == _NBLK - 1)
    def _():
        P = pooled_ref[...]
        pooled = P[:, 0:32] / (P[:, 32:33] + 1e-16)
        o = jnp.dot(jax.nn.relu(jnp.dot(pooled, w1_ref[...],
                                        preferred_element_type=_f32)
                                + b1_ref[...]),
                    w2_ref[...], preferred_element_type=_f32) + b2f_ref[...]
        out_ref[...] = o.reshape(1, G)


def _fin2(acc, batch3, bias2, Wg1, bg1, Wg2, bg2, W1, b1, W2, b2):
    return pl.pallas_call(
        _fin2_body,
        grid=(_NBLK,),
        in_specs=[
            pl.BlockSpec((2, _BLK, 48), lambda i: (0, i, 0)),
            pl.BlockSpec((1, 1, _BLK), lambda i: (i, 0, 0)),
            pl.BlockSpec((1, HID), lambda i: (0, 0)),
            pl.BlockSpec((HID, HID), lambda i: (0, 0)),
            pl.BlockSpec((1, HID), lambda i: (0, 0)),
            pl.BlockSpec((HID, 1), lambda i: (0, 0)),
            pl.BlockSpec((1, 1), lambda i: (0, 0)),
            pl.BlockSpec((HID, HID), lambda i: (0, 0)),
            pl.BlockSpec((1, HID), lambda i: (0, 0)),
            pl.BlockSpec((HID, 1), lambda i: (0, 0)),
            pl.BlockSpec((1, 1), lambda i: (0, 0)),
        ],
        out_specs=pl.BlockSpec((1, G), lambda i: (0, 0)),
        out_shape=jax.ShapeDtypeStruct((1, G), _f32),
        scratch_shapes=[pltpu.VMEM((G, 48), _f32)],
    )(acc, batch3, bias2, Wg1, bg1, Wg2, bg2, W1, b1, W2, b2)


def kernel(x, edge_index, batch, Wl1, bl1, Wr1, br1, att1, bias1,
           Wl2, bl2, Wr2, br2, att2, bias2,
           Wg1, bg1, Wg2, bg2, W1, b1, W2, b2):
    loop = jnp.arange(N, dtype=edge_index.dtype)
    padi = jnp.zeros((EP - EV,), edge_index.dtype)
    src = jnp.concatenate([edge_index[0], loop, padi])
    dst = jnp.concatenate([edge_index[1], loop, padi])
    xp = jnp.pad(x, ((0, NP - N), (0, 0)))
    att1b = att1.reshape(H1 * HID)
    att2b = att2.reshape(HID)
    batch3 = jnp.concatenate(
        [batch, jnp.full((NP - N,), G, batch.dtype)]).reshape(_NBLK, 1, _BLK)

    r2 = lambda v: v.reshape(1, -1)
    xl1, xr1 = _mm1(xp, Wl1, r2(bl1), Wr1, r2(br1))
    acc1 = _sc_edge_l1(src, dst, xl1, xr1, att1b)
    xl2, xr2 = _fin1(acc1, r2(bias1), Wl2, r2(bl2), Wr2, r2(br2))
    acc2 = _sc_edge_l2(src, dst, xl2, xr2, att2b)
    out = _fin2(acc2, batch3, r2(bias2), Wg1, r2(bg1), Wg2.reshape(HID, 1),
                r2(bg2), W1, r2(b1), W2.reshape(HID, 1), r2(b2))
    return out.reshape(G)


# speedup vs baseline: 64.4741x; 1.6015x over previous
import functools

import jax
import jax.numpy as jnp
from jax import lax
from jax.experimental import pallas as pl
from jax.experimental.pallas import tpu as pltpu
from jax.experimental.pallas import tpu_sc as plsc

N = 10000
E = 320000
D = 128
HID = 32
H1 = 2
G = 64

NP = 10240
EV = E + N
EP = 331776
NTILES = 32
EPT = EP // NTILES
K = 128
NCHUNKS = EPT // K
ROWS_PT = NP // 16

_f32 = jnp.float32
_i32 = jnp.int32


def _lane_take(v, idx):
    return lax.gather(
        v, idx[:, None],
        lax.GatherDimensionNumbers(
            offset_dims=(), collapsed_slice_dims=(0,), start_index_map=(0,)),
        slice_sizes=(1,),
        mode=lax.GatherScatterMode.PROMISE_IN_BOUNDS)


def _make_sc_edge(heads, outw):
    F = heads * HID

    mesh = plsc.VectorSubcoreMesh(
        core_axis_name="c", subcore_axis_name="s", num_cores=2,
        num_subcores=16)

    def body(src_hbm, dst_hbm, xl_hbm, xr_hbm, att_hbm, out_hbm,
             srcv, dstv, xj, xi, outb, attv, acc, sem1, sem2):
        c = lax.axis_index("c")
        s = lax.axis_index("s")
        lane = lax.iota(_i32, 16)

        pltpu.sync_copy(att_hbm, attv)

        def zrow(r, _):
            for q in range(outw // 16):
                outb[r, pl.ds(q * 16, 16)] = jnp.zeros((16,), _f32)
            return 0
        lax.fori_loop(0, K, zrow, 0)

        def zacc(j, _):
            pltpu.sync_copy(outb, acc.at[pl.ds(s * ROWS_PT + j * K, K)])
            return 0
        lax.fori_loop(0, ROWS_PT // K, zacc, 0)
        plsc.subcore_barrier()

        wbase = (c * 16 + s) * EPT

        def chunk(k, _):
            base = wbase + k * K
            pltpu.sync_copy(src_hbm.at[pl.ds(base, K)], srcv)
            pltpu.sync_copy(dst_hbm.at[pl.ds(base, K)], dstv)
            cp1 = pltpu.async_copy(xl_hbm.at[srcv], xj, sem1)
            cp2 = pltpu.async_copy(xr_hbm.at[dstv], xi, sem2)
            cp1.wait()
            cp2.wait()

            av = [attv[pl.ds(q * 16, 16)] for q in range(F // 16)]
            qph = HID // 16
            lanef = lane.astype(_f32)

            @plsc.parallel_loop(0, K, 1, unroll=8)
            def _edge(e):
                vjs = [xj[e, pl.ds(q * 16, 16)] for q in range(F // 16)]
                hsum = [jnp.zeros((16,), _f32) for _ in range(heads)]
                for q in range(F // 16):
                    t = vjs[q] + xi[e, pl.ds(q * 16, 16)]
                    lr = jnp.maximum(t, 0.2 * t)
                    hsum[q // qph] = hsum[q // qph] + lr * av[q]
                vf = jnp.clip(
                    jnp.full((16,), EV - (base + e), _i32).astype(_f32),
                    0.0, 1.0)
                tail = jnp.zeros((16,), _f32)
                for h in range(heads):
                    sv = hsum[h]
                    for kk in (8, 4, 2, 1):
                        sv = sv + _lane_take(sv, lane ^ kk)
                    pv = jnp.exp(sv) * vf
                    for q in range(h * qph, (h + 1) * qph):
                        outb[e, pl.ds(q * 16, 16)] = vjs[q] * pv
                    ohv = jnp.maximum(
                        1.0 - jnp.abs(lanef - float(h)), 0.0)
                    tail = tail + pv * ohv
                outb[e, pl.ds(F, 16)] = tail

            pltpu.sync_copy(outb, acc.at[dstv], add=True)
            return 0
        lax.fori_loop(0, NCHUNKS, chunk, 0)

        plsc.subcore_barrier()

        def wout(j, _):
            r = s * ROWS_PT + j * K
            pltpu.sync_copy(acc.at[pl.ds(r, K)], out_hbm.at[c, pl.ds(r, K)])
            return 0
        lax.fori_loop(0, ROWS_PT // K, wout, 0)

    return pl.kernel(
        body,
        out_type=jax.ShapeDtypeStruct((2, NP, outw), _f32),
        mesh=mesh,
        compiler_params=pltpu.CompilerParams(use_tc_tiling_on_sc=False),
        scratch_types=[
            pltpu.VMEM((K,), _i32),
            pltpu.VMEM((K,), _i32),
            pltpu.VMEM((K, F), _f32),
            pltpu.VMEM((K, F), _f32),
            pltpu.VMEM((K, outw), _f32),
            pltpu.VMEM((F,), _f32),
            pltpu.VMEM_SHARED((NP, outw), _f32),
            pltpu.SemaphoreType.DMA,
            pltpu.SemaphoreType.DMA,
        ],
    )


_sc_edge_l1 = _make_sc_edge(H1, 80)
_sc_edge_l2 = _make_sc_edge(1, 48)

_BLK = 512
_NBLK = NP // _BLK


def _mm1_body(x_ref, wl_ref, bl_ref, wr_ref, br_ref, xl_ref, xr_ref):
    xb = x_ref[...]
    xl_ref[...] = jnp.dot(xb, wl_ref[...],
                          preferred_element_type=_f32) + bl_ref[...]
    xr_ref[...] = jnp.dot(xb, wr_ref[...],
                          preferred_element_type=_f32) + br_ref[...]


def _mm1(xp, Wl1, bl1, Wr1, br1):
    return pl.pallas_call(
        _mm1_body,
        grid=(_NBLK,),
        in_specs=[
            pl.BlockSpec((_BLK, D), lambda i: (i, 0)),
            pl.BlockSpec((D, H1 * HID), lambda i: (0, 0)),
            pl.BlockSpec((1, H1 * HID), lambda i: (0, 0)),
            pl.BlockSpec((D, H1 * HID), lambda i: (0, 0)),
            pl.BlockSpec((1, H1 * HID), lambda i: (0, 0)),
        ],
        out_specs=[
            pl.BlockSpec((_BLK, H1 * HID), lambda i: (i, 0)),
            pl.BlockSpec((_BLK, H1 * HID), lambda i: (i, 0)),
        ],
        out_shape=[
            jax.ShapeDtypeStruct((NP, H1 * HID), _f32),
            jax.ShapeDtypeStruct((NP, H1 * HID), _f32),
        ],
    )(xp, Wl1, bl1, Wr1, br1)


def _fin1_body(acc_ref, b1_ref, wl_ref, bl_ref, wr_ref, br_ref,
               xl_ref, xr_ref):
    a = acc_ref[0] + acc_ref[1]
    s0 = a[:, 64:65] + 1e-16
    s1 = a[:, 65:66] + 1e-16
    h = jnp.concatenate([a[:, 0:32] / s0, a[:, 32:64] / s1], axis=1)
    h = jax.nn.relu(h + b1_ref[...])
    xl_ref[...] = jnp.dot(h, wl_ref[...],
                          preferred_element_type=_f32) + bl_ref[...]
    xr_ref[...] = jnp.dot(h, wr_ref[...],
                          preferred_element_type=_f32) + br_ref[...]


def _fin1(acc, bias1, Wl2, bl2, Wr2, br2):
    return pl.pallas_call(
        _fin1_body,
        grid=(_NBLK,),
        in_specs=[
            pl.BlockSpec((2, _BLK, 80), lambda i: (0, i, 0)),
            pl.BlockSpec((1, H1 * HID), lambda i: (0, 0)),
            pl.BlockSpec((H1 * HID, HID), lambda i: (0, 0)),
            pl.BlockSpec((1, HID), lambda i: (0, 0)),
            pl.BlockSpec((H1 * HID, HID), lambda i: (0, 0)),
            pl.BlockSpec((1, HID), lambda i: (0, 0)),
        ],
        out_specs=[
            pl.BlockSpec((_BLK, HID), lambda i: (i, 0)),
            pl.BlockSpec((_BLK, HID), lambda i: (i, 0)),
        ],
        out_shape=[
            jax.ShapeDtypeStruct((NP, HID), _f32),
            jax.ShapeDtypeStruct((NP, HID), _f32),
        ],
    )(acc, bias1, Wl2, bl2, Wr2, br2)


def _fin2_body(acc_ref, batch_ref, b2_ref, wg1_ref, bg1_ref, wg2_ref,
               bg2_ref, w1_ref, b1_ref, w2_ref, b2f_ref, out_ref,
               pooled_ref):
    i = pl.program_id(0)

    @pl.when(i == 0)
    def _():
        pooled_ref[...] = jnp.zeros((G, 48), _f32)

    a = acc_ref[0] + acc_ref[1]
    s = a[:, 32:33] + 1e-16
    h2 = jax.nn.relu(a[:, 0:32] / s + b2_ref[...])
    gate = jnp.dot(jax.nn.relu(jnp.dot(h2, wg1_ref[...],
                                       preferred_element_type=_f32)
                               + bg1_ref[...]),
                   wg2_ref[...], preferred_element_type=_f32) + bg2_ref[...]
    w = jnp.exp(gate[:, 0])
    bidx = batch_ref[0, 0, :].astype(_f32)
    gi = lax.broadcasted_iota(_i32, (G, _BLK), 0).astype(_f32)
    oh = jnp.maximum(1.0 - jnp.abs(gi - bidx[None, :]), 0.0)
    ohw = oh * w[None, :]
    feat = jnp.concatenate(
        [h2, jnp.ones((_BLK, 1), _f32), jnp.zeros((_BLK, 15), _f32)], axis=1)
    pooled_ref[...] += jnp.dot(ohw, feat, preferred_element_type=_f32)

    @pl.when(i == _NBLK - 1)
    def _():
        P = pooled_ref[...]
        pooled = P[:, 0:32] / (P[:, 32:33] + 1e-16)
        o = jnp.dot(jax.nn.relu(jnp.dot(pooled, w1_ref[...],
                                        preferred_element_type=_f32)
                                + b1_ref[...]),
                    w2_ref[...], preferred_element_type=_f32) + b2f_ref[...]
        out_ref[...] = o.reshape(1, G)


def _fin2(acc, batch3, bias2, Wg1, bg1, Wg2, bg2, W1, b1, W2, b2):
    return pl.pallas_call(
        _fin2_body,
        grid=(_NBLK,),
        in_specs=[
            pl.BlockSpec((2, _BLK, 48), lambda i: (0, i, 0)),
            pl.BlockSpec((1, 1, _BLK), lambda i: (i, 0, 0)),
            pl.BlockSpec((1, HID), lambda i: (0, 0)),
            pl.BlockSpec((HID, HID), lambda i: (0, 0)),
            pl.BlockSpec((1, HID), lambda i: (0, 0)),
            pl.BlockSpec((HID, 1), lambda i: (0, 0)),
            pl.BlockSpec((1, 1), lambda i: (0, 0)),
            pl.BlockSpec((HID, HID), lambda i: (0, 0)),
            pl.BlockSpec((1, HID), lambda i: (0, 0)),
            pl.BlockSpec((HID, 1), lambda i: (0, 0)),
            pl.BlockSpec((1, 1), lambda i: (0, 0)),
        ],
        out_specs=pl.BlockSpec((1, G), lambda i: (0, 0)),
        out_shape=jax.ShapeDtypeStruct((1, G), _f32),
        scratch_shapes=[pltpu.VMEM((G, 48), _f32)],
    )(acc, batch3, bias2, Wg1, bg1, Wg2, bg2, W1, b1, W2, b2)


def kernel(x, edge_index, batch, Wl1, bl1, Wr1, br1, att1, bias1,
           Wl2, bl2, Wr2, br2, att2, bias2,
           Wg1, bg1, Wg2, bg2, W1, b1, W2, b2):
    loop = jnp.arange(N, dtype=edge_index.dtype)
    padi = jnp.zeros((EP - EV,), edge_index.dtype)
    src = jnp.concatenate([edge_index[0], loop, padi])
    dst = jnp.concatenate([edge_index[1], loop, padi])
    xp = jnp.pad(x, ((0, NP - N), (0, 0)))
    att1b = att1.reshape(H1 * HID)
    att2b = att2.reshape(HID)
    batch3 = jnp.concatenate(
        [batch, jnp.full((NP - N,), G, batch.dtype)]).reshape(_NBLK, 1, _BLK)

    r2 = lambda v: v.reshape(1, -1)
    xl1, xr1 = _mm1(xp, Wl1, r2(bl1), Wr1, r2(br1))
    acc1 = _sc_edge_l1(src, dst, xl1, xr1, att1b)
    xl2, xr2 = _fin1(acc1, r2(bias1), Wl2, r2(bl2), Wr2, r2(br2))
    acc2 = _sc_edge_l2(src, dst, xl2, xr2, att2b)
    out = _fin2(acc2, batch3, r2(bias2), Wg1, r2(bg1), Wg2.reshape(HID, 1),
                r2(bg2), W1, r2(b1), W2.reshape(HID, 1), r2(b2))
    return out.reshape(G)


# speedup vs baseline: 65.0789x; 1.0094x over previous
import functools

import jax
import jax.numpy as jnp
from jax import lax
from jax.experimental import pallas as pl
from jax.experimental.pallas import tpu as pltpu
from jax.experimental.pallas import tpu_sc as plsc

N = 10000
E = 320000
D = 128
HID = 32
H1 = 2
G = 64

NP = 10240
EV = E + N
NTILES = 32
K = 128
NCHUNKS = 82
EPT = NCHUNKS * K
EP = EPT * NTILES
ROWS_PT = NP // 16

_f32 = jnp.float32
_i32 = jnp.int32


def _lane_take(v, idx):
    return lax.gather(
        v, idx[:, None],
        lax.GatherDimensionNumbers(
            offset_dims=(), collapsed_slice_dims=(0,), start_index_map=(0,)),
        slice_sizes=(1,),
        mode=lax.GatherScatterMode.PROMISE_IN_BOUNDS)


def _make_sc_edge(heads, outw):
    F = heads * HID

    mesh = plsc.VectorSubcoreMesh(
        core_axis_name="c", subcore_axis_name="s", num_cores=2,
        num_subcores=16)

    def body(src_hbm, dst_hbm, xl_hbm, xr_hbm, att_hbm, out_hbm,
             srcv0, dstv0, xj0, xi0, outb0, srcv1, dstv1, xj1, xi1, outb1,
             attv, acc, semj0, semi0, semc0, semj1, semi1, semc1):
        c = lax.axis_index("c")
        s = lax.axis_index("s")
        lane = lax.iota(_i32, 16)
        srcv = (srcv0, srcv1)
        dstv = (dstv0, dstv1)
        xj = (xj0, xj1)
        xi = (xi0, xi1)
        outb = (outb0, outb1)
        semj = (semj0, semj1)
        semi = (semi0, semi1)
        semc = (semc0, semc1)

        pltpu.sync_copy(att_hbm, attv)

        @plsc.parallel_loop(0, K, 1, unroll=4)
        def _zrow(r):
            for q in range(outw // 16):
                outb0[r, pl.ds(q * 16, 16)] = jnp.zeros((16,), _f32)

        def zacc(j, _):
            pltpu.sync_copy(outb0, acc.at[pl.ds(s * ROWS_PT + j * K, K)])
            return 0
        lax.fori_loop(0, ROWS_PT // K, zacc, 0)
        plsc.subcore_barrier()

        wbase = (c * 16 + s) * EPT
        av = [attv[pl.ds(q * 16, 16)] for q in range(F // 16)]
        qph = HID // 16
        lanef = lane.astype(_f32)

        def fetch(k, b):
            base = wbase + k * K
            pltpu.sync_copy(src_hbm.at[pl.ds(base, K)], srcv[b])
            pltpu.sync_copy(dst_hbm.at[pl.ds(base, K)], dstv[b])
            pltpu.async_copy(xl_hbm.at[srcv[b]], xj[b], semj[b])
            pltpu.async_copy(xr_hbm.at[dstv[b]], xi[b], semi[b])

        def compute(k, b):
            base = wbase + k * K

            @plsc.parallel_loop(0, K, 1, unroll=8)
            def _edge(e):
                vjs = [xj[b][e, pl.ds(q * 16, 16)] for q in range(F // 16)]
                hsum = [jnp.zeros((16,), _f32) for _ in range(heads)]
                for q in range(F // 16):
                    t = vjs[q] + xi[b][e, pl.ds(q * 16, 16)]
                    lr = jnp.maximum(t, 0.2 * t)
                    hsum[q // qph] = hsum[q // qph] + lr * av[q]
                vf = jnp.clip(
                    jnp.full((16,), EV - (base + e), _i32).astype(_f32),
                    0.0, 1.0)
                tail = jnp.zeros((16,), _f32)
                for h in range(heads):
                    sv = hsum[h]
                    for kk in (8, 4, 2, 1):
                        sv = sv + _lane_take(sv, lane ^ kk)
                    pv = jnp.exp(sv) * vf
                    for q in range(h * qph, (h + 1) * qph):
                        outb[b][e, pl.ds(q * 16, 16)] = vjs[q] * pv
                    ohv = jnp.maximum(
                        1.0 - jnp.abs(lanef - float(h)), 0.0)
                    tail = tail + pv * ohv
                outb[b][e, pl.ds(F, 16)] = tail

        def halfstep(k, b):
            pltpu.make_async_copy(xl_hbm.at[srcv[b]], xj[b], semj[b]).wait()
            pltpu.make_async_copy(xr_hbm.at[dstv[b]], xi[b], semi[b]).wait()
            @pl.when(k >= 1)
            def _():
                pltpu.make_async_copy(
                    outb[1 - b], acc.at[dstv[1 - b]], semc[1 - b]).wait()

            @pl.when(k + 1 < NCHUNKS)
            def _():
                fetch(k + 1, 1 - b)
            compute(k, b)
            pltpu.async_copy(outb[b], acc.at[dstv[b]], semc[b], add=True)

        fetch(0, 0)

        def pair(j, _):
            halfstep(2 * j, 0)
            halfstep(2 * j + 1, 1)
            return 0
        lax.fori_loop(0, NCHUNKS // 2, pair, 0)

        pltpu.make_async_copy(outb[1], acc.at[dstv[1]], semc[1]).wait()
        plsc.subcore_barrier()

        def wout(j, _):
            r = s * ROWS_PT + j * K
            pltpu.sync_copy(acc.at[pl.ds(r, K)], out_hbm.at[c, pl.ds(r, K)])
            return 0
        lax.fori_loop(0, ROWS_PT // K, wout, 0)

    return pl.kernel(
        body,
        out_type=jax.ShapeDtypeStruct((2, NP, outw), _f32),
        mesh=mesh,
        compiler_params=pltpu.CompilerParams(use_tc_tiling_on_sc=False),
        scratch_types=(
            [pltpu.VMEM((K,), _i32),
             pltpu.VMEM((K,), _i32),
             pltpu.VMEM((K, F), _f32),
             pltpu.VMEM((K, F), _f32),
             pltpu.VMEM((K, outw), _f32)] * 2
            + [pltpu.VMEM((F,), _f32),
               pltpu.VMEM_SHARED((NP, outw), _f32)]
            + [pltpu.SemaphoreType.DMA] * 6
        ),
    )


_sc_edge_l1 = _make_sc_edge(H1, 80)
_sc_edge_l2 = _make_sc_edge(1, 48)

_BLK = 512
_NBLK = NP // _BLK


def _mm1_body(x_ref, wl_ref, bl_ref, wr_ref, br_ref, xl_ref, xr_ref):
    xb = x_ref[...]
    xl_ref[...] = jnp.dot(xb, wl_ref[...],
                          preferred_element_type=_f32) + bl_ref[...]
    xr_ref[...] = jnp.dot(xb, wr_ref[...],
                          preferred_element_type=_f32) + br_ref[...]


def _mm1(xp, Wl1, bl1, Wr1, br1):
    return pl.pallas_call(
        _mm1_body,
        grid=(_NBLK,),
        in_specs=[
            pl.BlockSpec((_BLK, D), lambda i: (i, 0)),
            pl.BlockSpec((D, H1 * HID), lambda i: (0, 0)),
            pl.BlockSpec((1, H1 * HID), lambda i: (0, 0)),
            pl.BlockSpec((D, H1 * HID), lambda i: (0, 0)),
            pl.BlockSpec((1, H1 * HID), lambda i: (0, 0)),
        ],
        out_specs=[
            pl.BlockSpec((_BLK, H1 * HID), lambda i: (i, 0)),
            pl.BlockSpec((_BLK, H1 * HID), lambda i: (i, 0)),
        ],
        out_shape=[
            jax.ShapeDtypeStruct((NP, H1 * HID), _f32),
            jax.ShapeDtypeStruct((NP, H1 * HID), _f32),
        ],
    )(xp, Wl1, bl1, Wr1, br1)


def _fin1_body(acc_ref, b1_ref, wl_ref, bl_ref, wr_ref, br_ref,
               xl_ref, xr_ref):
    a = acc_ref[0] + acc_ref[1]
    s0 = a[:, 64:65] + 1e-16
    s1 = a[:, 65:66] + 1e-16
    h = jnp.concatenate([a[:, 0:32] / s0, a[:, 32:64] / s1], axis=1)
    h = jax.nn.relu(h + b1_ref[...])
    xl_ref[...] = jnp.dot(h, wl_ref[...],
                          preferred_element_type=_f32) + bl_ref[...]
    xr_ref[...] = jnp.dot(h, wr_ref[...],
                          preferred_element_type=_f32) + br_ref[...]


def _fin1(acc, bias1, Wl2, bl2, Wr2, br2):
    return pl.pallas_call(
        _fin1_body,
        grid=(_NBLK,),
        in_specs=[
            pl.BlockSpec((2, _BLK, 80), lambda i: (0, i, 0)),
            pl.BlockSpec((1, H1 * HID), lambda i: (0, 0)),
            pl.BlockSpec((H1 * HID, HID), lambda i: (0, 0)),
            pl.BlockSpec((1, HID), lambda i: (0, 0)),
            pl.BlockSpec((H1 * HID, HID), lambda i: (0, 0)),
            pl.BlockSpec((1, HID), lambda i: (0, 0)),
        ],
        out_specs=[
            pl.BlockSpec((_BLK, HID), lambda i: (i, 0)),
            pl.BlockSpec((_BLK, HID), lambda i: (i, 0)),
        ],
        out_shape=[
            jax.ShapeDtypeStruct((NP, HID), _f32),
            jax.ShapeDtypeStruct((NP, HID), _f32),
        ],
    )(acc, bias1, Wl2, bl2, Wr2, br2)


def _fin2_body(acc_ref, batch_ref, b2_ref, wg1_ref, bg1_ref, wg2_ref,
               bg2_ref, w1_ref, b1_ref, w2_ref, b2f_ref, out_ref,
               pooled_ref):
    i = pl.program_id(0)

    @pl.when(i == 0)
    def _():
        pooled_ref[...] = jnp.zeros((G, 48), _f32)

    a = acc_ref[0] + acc_ref[1]
    s = a[:, 32:33] + 1e-16
    h2 = jax.nn.relu(a[:, 0:32] / s + b2_ref[...])
    gate = jnp.dot(jax.nn.relu(jnp.dot(h2, wg1_ref[...],
                                       preferred_element_type=_f32)
                               + bg1_ref[...]),
                   wg2_ref[...], preferred_element_type=_f32) + bg2_ref[...]
    w = jnp.exp(gate[:, 0])
    bidx = batch_ref[0, 0, :].astype(_f32)
    gi = lax.broadcasted_iota(_i32, (G, _BLK), 0).astype(_f32)
    oh = jnp.maximum(1.0 - jnp.abs(gi - bidx[None, :]), 0.0)
    ohw = oh * w[None, :]
    feat = jnp.concatenate(
        [h2, jnp.ones((_BLK, 1), _f32), jnp.zeros((_BLK, 15), _f32)], axis=1)
    pooled_ref[...] += jnp.dot(ohw, feat, preferred_element_type=_f32)

    @pl.when(i == _NBLK - 1)
    def _():
        P = pooled_ref[...]
        pooled = P[:, 0:32] / (P[:, 32:33] + 1e-16)
        o = jnp.dot(jax.nn.relu(jnp.dot(pooled, w1_ref[...],
                                        preferred_element_type=_f32)
                                + b1_ref[...]),
                    w2_ref[...], preferred_element_type=_f32) + b2f_ref[...]
        out_ref[...] = o.reshape(1, G)


def _fin2(acc, batch3, bias2, Wg1, bg1, Wg2, bg2, W1, b1, W2, b2):
    return pl.pallas_call(
        _fin2_body,
        grid=(_NBLK,),
        in_specs=[
            pl.BlockSpec((2, _BLK, 48), lambda i: (0, i, 0)),
            pl.BlockSpec((1, 1, _BLK), lambda i: (i, 0, 0)),
            pl.BlockSpec((1, HID), lambda i: (0, 0)),
            pl.BlockSpec((HID, HID), lambda i: (0, 0)),
            pl.BlockSpec((1, HID), lambda i: (0, 0)),
            pl.BlockSpec((HID, 1), lambda i: (0, 0)),
            pl.BlockSpec((1, 1), lambda i: (0, 0)),
            pl.BlockSpec((HID, HID), lambda i: (0, 0)),
            pl.BlockSpec((1, HID), lambda i: (0, 0)),
            pl.BlockSpec((HID, 1), lambda i: (0, 0)),
            pl.BlockSpec((1, 1), lambda i: (0, 0)),
        ],
        out_specs=pl.BlockSpec((1, G), lambda i: (0, 0)),
        out_shape=jax.ShapeDtypeStruct((1, G), _f32),
        scratch_shapes=[pltpu.VMEM((G, 48), _f32)],
    )(acc, batch3, bias2, Wg1, bg1, Wg2, bg2, W1, b1, W2, b2)


def kernel(x, edge_index, batch, Wl1, bl1, Wr1, br1, att1, bias1,
           Wl2, bl2, Wr2, br2, att2, bias2,
           Wg1, bg1, Wg2, bg2, W1, b1, W2, b2):
    loop = jnp.arange(N, dtype=edge_index.dtype)
    padi = jnp.zeros((EP - EV,), edge_index.dtype)
    src = jnp.concatenate([edge_index[0], loop, padi])
    dst = jnp.concatenate([edge_index[1], loop, padi])
    xp = jnp.pad(x, ((0, NP - N), (0, 0)))
    att1b = att1.reshape(H1 * HID)
    att2b = att2.reshape(HID)
    batch3 = jnp.concatenate(
        [batch, jnp.full((NP - N,), G, batch.dtype)]).reshape(_NBLK, 1, _BLK)

    r2 = lambda v: v.reshape(1, -1)
    xl1, xr1 = _mm1(xp, Wl1, r2(bl1), Wr1, r2(br1))
    acc1 = _sc_edge_l1(src, dst, xl1, xr1, att1b)
    xl2, xr2 = _fin1(acc1, r2(bias1), Wl2, r2(bl2), Wr2, r2(br2))
    acc2 = _sc_edge_l2(src, dst, xl2, xr2, att2b)
    out = _fin2(acc2, batch3, r2(bias2), Wg1, r2(bg1), Wg2.reshape(HID, 1),
                r2(bg2), W1, r2(b1), W2.reshape(HID, 1), r2(b2))
    return out.reshape(G)
